# baseline (speedup 1.0000x reference)
import jax
import jax.numpy as jnp
from jax.experimental import pallas as pl
from jax.experimental.pallas import tpu as pltpu

N = 4096
BM = 512

def _probe_kernel(adj_ref, o_ref, a16_ref):
    i = pl.program_id(0)
    rows = pl.ds(i * BM, BM)
    a16 = adj_ref[...].astype(jnp.bfloat16)
    a16_ref[rows, :] = a16
    o_ref[...] = adj_ref[:, :128]

def kernel(x, adj, W1, b1, W2, b2, W3, b3, W4, b4):
    return pl.pallas_call(
        _probe_kernel,
        grid=(N // BM,),
        in_specs=[pl.BlockSpec((BM, N), lambda i: (i, 0))],
        out_specs=pl.BlockSpec((BM, 128), lambda i: (i, 0)),
        out_shape=jax.ShapeDtypeStruct((N, 128), jnp.float32),
        scratch_shapes=[pltpu.VMEM((N, N), jnp.bfloat16)],
        compiler_params=pltpu.CompilerParams(dimension_semantics=("arbitrary",)),
    )(adj)
